# baseline (speedup 1.0000x reference)
import functools

import jax
import jax.numpy as jnp
from jax import lax
from jax.experimental import pallas as pl
from jax.experimental.pallas import tpu as pltpu
from jax.experimental.pallas import tpu_sc as plsc

B = 4
N = 10000
M = 2500
MP = 2560
NN = 16
K = 3
A = 4
C = 64
O = 64
D = A * C
TOT = B * MP
KD = K * D

NW = 32
PER_W = TOT // NW
NB = 4
NCH = PER_W // NB
ROWS = NB * NN
PAD = 64


def _sc_wsum(table, idx_flat, w_flat):
    mesh = plsc.VectorSubcoreMesh(core_axis_name="c", subcore_axis_name="s")
    info = plsc.get_sparse_core_info()
    nc = info.num_cores

    @functools.partial(
        pl.kernel,
        mesh=mesh,
        out_type=jax.ShapeDtypeStruct((TOT, KD), jnp.float32),
        scratch_types=[
            pltpu.VMEM((PER_W * NN,), jnp.int32),
            pltpu.VMEM((PER_W * K * 16,), jnp.float32),
            pltpu.VMEM((ROWS, D), jnp.float32),
            pltpu.VMEM((ROWS, D), jnp.float32),
            pltpu.VMEM((NB, KD), jnp.float32),
            pltpu.VMEM((NB, KD), jnp.float32),
            pltpu.VMEM((NB, KD), jnp.float32),
            pltpu.SemaphoreType.DMA,
            pltpu.SemaphoreType.DMA,
            pltpu.SemaphoreType.DMA,
            pltpu.SemaphoreType.DMA,
        ],
    )
    def k(table_hbm, idx_hbm, w_hbm, out_hbm,
          idx_all, w_all, rows_v0, rows_v1, out_v0, out_v1, zer_v,
          semg0, semg1, semo0, semo1):
        wid = lax.axis_index("s") * nc + lax.axis_index("c")
        base = wid * PER_W
        bq = base // MP
        mm0 = base - bq * MP
        rbase = bq * M + mm0
        nvalid = (jnp.minimum(M - mm0, PER_W)) // NB
        row_bufs = (rows_v0, rows_v1)
        out_bufs = (out_v0, out_v1)
        semg = (semg0, semg1)
        semo = (semo0, semo1)

        pltpu.sync_copy(idx_hbm.at[pl.ds(rbase * NN, PER_W * NN)], idx_all)
        pltpu.sync_copy(w_hbm.at[pl.ds(rbase * K * 16, PER_W * K * 16)],
                        w_all)

        for j in range(NB):
            for t in range(KD // 16):
                zer_v[j, pl.ds(t * 16, 16)] = jnp.zeros((16,), jnp.float32)

        def gather_start(c, b):
            @pl.when(c < nvalid)
            def _():
                pltpu.make_async_copy(
                    table_hbm.at[idx_all.at[pl.ds(c * ROWS, ROWS)]],
                    row_bufs[b], semg[b]).start()

        gather_start(0, 0)
        gather_start(1, 1)

        def compute_chunk(rows_v, out_v, c):
            for j in range(NB):
                woff = (c * NB + j) * (K * 16)
                wv = [w_all[pl.ds(woff + kk * 16, 16)] for kk in range(K)]
                for dg in range(4):
                    accs = [[jnp.zeros((16,), jnp.float32)
                             for _ in range(4)] for _ in range(K)]
                    for n in range(16):
                        r = [rows_v[j * NN + n, pl.ds(dg * 64 + d * 16, 16)]
                             for d in range(4)]
                        for kk in range(K):
                            wspl = wv[kk].at[
                                jnp.full((16,), n, jnp.int32)
                            ].get(mode="promise_in_bounds")
                            for d in range(4):
                                accs[kk][d] = accs[kk][d] + wspl * r[d]
                    for kk in range(K):
                        for d in range(4):
                            out_v[j, pl.ds(kk * D + dg * 64 + d * 16,
                                           16)] = accs[kk][d]

        def super_body(s, carry):
            for b in range(2):
                c = s * 2 + b
                v0 = base + c * NB
                valid = c < nvalid

                @pl.when(jnp.logical_not(valid))
                def _():
                    pltpu.sync_copy(zer_v, out_hbm.at[pl.ds(v0, NB)])

                @pl.when(valid)
                def _():
                    pltpu.make_async_copy(
                        table_hbm.at[idx_all.at[pl.ds(c * ROWS, ROWS)]],
                        row_bufs[b], semg[b]).wait()

                    @pl.when(c >= 2)
                    def _():
                        pltpu.make_async_copy(
                            out_bufs[b], out_hbm.at[pl.ds(v0, NB)],
                            semo[b]).wait()

                    compute_chunk(row_bufs[b], out_bufs[b], c)
                    pltpu.make_async_copy(
                        out_bufs[b], out_hbm.at[pl.ds(v0, NB)],
                        semo[b]).start()
                    gather_start(c + 2, b)
            return carry

        lax.fori_loop(0, NCH // 2, super_body, 0)

        for b in range(2):
            pltpu.make_async_copy(
                out_bufs[b], out_hbm.at[pl.ds(base, NB)], semo[b]).wait()

    return k(table, idx_flat, w_flat)


def _tr_body(f_ref, t_ref):
    x = f_ref[0].reshape(D // 2, N)
    t_ref[...] = jnp.transpose(x, (1, 0))


def _tc_table(feats):
    return pl.pallas_call(
        _tr_body,
        grid=(B, 2),
        in_specs=[pl.BlockSpec((1, C // 2, A, N), lambda b, j: (b, j, 0, 0))],
        out_specs=pl.BlockSpec((N, D // 2), lambda b, j: (b, j)),
        out_shape=jax.ShapeDtypeStruct((B * N, D), jnp.float32),
    )(feats)


_RB = 512
_GRID = TOT // _RB


def _mm_body(x_ref, wt_ref, y_ref, sum_ref, sq_ref, w2_ref):
    @pl.when(pl.program_id(0) == 0)
    def _init():
        sum_ref[...] = jnp.zeros_like(sum_ref)
        sq_ref[...] = jnp.zeros_like(sq_ref)
        r4 = (lax.broadcasted_iota(jnp.int32, (D, C), 0) // A
              == lax.broadcasted_iota(jnp.int32, (D, C), 1)
              ).astype(jnp.float32)
        mask = (lax.broadcasted_iota(jnp.int32, (D, D), 0) % A
                == lax.broadcasted_iota(jnp.int32, (D, D), 1) % A
                ).astype(jnp.float32)
        for k in range(K):
            rep = jnp.dot(jnp.dot(r4, wt_ref[k],
                                  preferred_element_type=jnp.float32),
                          jnp.transpose(r4, (1, 0)),
                          preferred_element_type=jnp.float32)
            w2_ref[k] = rep * mask

    y = jnp.dot(x_ref[:, 0:D], w2_ref[0], preferred_element_type=jnp.float32)
    for k in range(1, K):
        y += jnp.dot(x_ref[:, k * D:(k + 1) * D], w2_ref[k],
                     preferred_element_type=jnp.float32)
    y_ref[...] = y
    sum_ref[...] += jnp.sum(y, axis=0, keepdims=True)
    sq_ref[...] += jnp.sum(y * y, axis=0, keepdims=True)


def _tc_matmul_stats(x, wt):
    return pl.pallas_call(
        _mm_body,
        grid=(_GRID,),
        in_specs=[
            pl.BlockSpec((_RB, KD), lambda i: (i, 0)),
            pl.BlockSpec((K, C, O), lambda i: (0, 0, 0)),
        ],
        out_specs=[
            pl.BlockSpec((_RB, D), lambda i: (i, 0)),
            pl.BlockSpec((1, D), lambda i: (0, 0)),
            pl.BlockSpec((1, D), lambda i: (0, 0)),
        ],
        out_shape=[
            jax.ShapeDtypeStruct((TOT, D), jnp.float32),
            jax.ShapeDtypeStruct((1, D), jnp.float32),
            jax.ShapeDtypeStruct((1, D), jnp.float32),
        ],
        scratch_shapes=[pltpu.VMEM((K, D, D), jnp.float32)],
    )(x, wt)


def _bn_body(y_ref, sc_ref, sh_ref, o_ref):
    z = jnp.maximum(y_ref[...] * sc_ref[...] + sh_ref[...], 0.0)
    t = jnp.transpose(z, (1, 0))
    o_ref[0] = t.reshape(O, A, MP)[:, :, :M]


def _tc_bn_relu(y, scale_col, shift_col):
    return pl.pallas_call(
        _bn_body,
        grid=(B,),
        in_specs=[
            pl.BlockSpec((MP, D), lambda b: (b, 0)),
            pl.BlockSpec((1, D), lambda b: (0, 0)),
            pl.BlockSpec((1, D), lambda b: (0, 0)),
        ],
        out_specs=pl.BlockSpec((1, O, A, M), lambda b: (b, 0, 0, 0)),
        out_shape=jax.ShapeDtypeStruct((B, O, A, M), jnp.float32),
    )(y, scale_col, shift_col)


def kernel(xyz, feats, inter_idx, inter_w, W, gamma, beta):
    table = _tc_table(feats)
    idxp = (inter_idx.astype(jnp.int32)
            + (jnp.arange(B, dtype=jnp.int32) * N)[:, None, None])
    idx_flat = jnp.concatenate(
        [idxp.reshape(B * M * NN), jnp.zeros((PAD * NN,), jnp.int32)])
    w_flat = jnp.concatenate(
        [inter_w.reshape(B * M * K * NN),
         jnp.zeros((PAD * K * NN,), jnp.float32)])
    wt = W.transpose(2, 1, 0)

    wsum = _sc_wsum(table, idx_flat, w_flat)

    y, colsum, colsq = _tc_matmul_stats(wsum, wt)

    cnt = jnp.float32(B * M * A)
    mean = colsum.reshape(O, A).sum(1) / cnt
    var = colsq.reshape(O, A).sum(1) / cnt - mean * mean
    scale_o = gamma * lax.rsqrt(var + 1e-5)
    shift_o = beta - mean * scale_o
    scale_col = jnp.repeat(scale_o, A).reshape(1, D)
    shift_col = jnp.repeat(shift_o, A).reshape(1, D)

    feat = _tc_bn_relu(y, scale_col, shift_col)
    new_xyz = xyz[:, :, ::4]
    return (inter_idx, inter_w, new_xyz, feat)


# speedup vs baseline: 1.4694x; 1.4694x over previous
import functools

import jax
import jax.numpy as jnp
from jax import lax
from jax.experimental import pallas as pl
from jax.experimental.pallas import tpu as pltpu
from jax.experimental.pallas import tpu_sc as plsc

B = 4
N = 10000
M = 2500
MP = 2560
NN = 16
K = 3
A = 4
C = 64
O = 64
D = A * C
TOT = B * MP
KD = K * D

NW = 32
PER_W = TOT // NW
NB = 4
NCH = PER_W // NB
ROWS = NB * NN


def _sc_wsum(table, idx_flat, w_flat):
    mesh = plsc.VectorSubcoreMesh(core_axis_name="c", subcore_axis_name="s")
    info = plsc.get_sparse_core_info()
    nc = info.num_cores

    @functools.partial(
        pl.kernel,
        mesh=mesh,
        out_type=jax.ShapeDtypeStruct((TOT, KD), jnp.float32),
        scratch_types=[
            pltpu.VMEM((ROWS,), jnp.int32),
            pltpu.VMEM((ROWS,), jnp.int32),
            pltpu.VMEM((NB, K, 16), jnp.float32),
            pltpu.VMEM((NB, K, 16), jnp.float32),
            pltpu.VMEM((ROWS, D), jnp.float32),
            pltpu.VMEM((ROWS, D), jnp.float32),
            pltpu.VMEM((NB, KD), jnp.float32),
            pltpu.VMEM((NB, KD), jnp.float32),
            pltpu.SemaphoreType.DMA,
            pltpu.SemaphoreType.DMA,
        ],
    )
    def k(table_hbm, idx_hbm, w_hbm, out_hbm,
          idx_v0, idx_v1, w_v0, w_v1, rows_v0, rows_v1, out_v, zer_v,
          sem0, sem1):
        wid = lax.axis_index("s") * nc + lax.axis_index("c")
        base = wid * PER_W
        idx_bufs = (idx_v0, idx_v1)
        w_bufs = (w_v0, w_v1)
        row_bufs = (rows_v0, rows_v1)
        sems = (sem0, sem1)

        for j in range(NB):
            for t in range(KD // 16):
                zer_v[j, pl.ds(t * 16, 16)] = jnp.zeros((16,), jnp.float32)

        def vmap_chunk(c):
            v0 = base + c * NB
            bq = v0 // MP
            mm = v0 - bq * MP
            return mm < M, bq * M + mm

        def stage(c, b):
            valid, r0 = vmap_chunk(c)

            @pl.when(valid)
            def _():
                pltpu.sync_copy(idx_hbm.at[pl.ds(r0 * NN, ROWS)], idx_bufs[b])
                pltpu.sync_copy(w_hbm.at[pl.ds(r0, NB)], w_bufs[b])
                pltpu.make_async_copy(
                    table_hbm.at[idx_bufs[b]], row_bufs[b], sems[b]).start()

        stage(0, 0)
        stage(1, 1)

        def compute_chunk(rows_v, w_v, v0):
            def j_body(j, carry2):
                wv = [w_v[j, kk, :] for kk in range(K)]
                for dg in range(4):
                    accs = [[jnp.zeros((16,), jnp.float32)
                             for _ in range(4)] for _ in range(K)]
                    for n in range(16):
                        r = [rows_v[j * NN + n, pl.ds(dg * 64 + d * 16, 16)]
                             for d in range(4)]
                        for kk in range(K):
                            wspl = wv[kk].at[
                                jnp.full((16,), n, jnp.int32)
                            ].get(mode="promise_in_bounds")
                            for d in range(4):
                                accs[kk][d] = accs[kk][d] + wspl * r[d]
                    for kk in range(K):
                        for d in range(4):
                            out_v[j, pl.ds(kk * D + dg * 64 + d * 16,
                                           16)] = accs[kk][d]
                return carry2

            lax.fori_loop(0, NB, j_body, 0)
            pltpu.sync_copy(out_v, out_hbm.at[pl.ds(v0, NB)])

        def super_body(s, carry):
            for b in range(2):
                c = s * 2 + b
                valid, _r0 = vmap_chunk(c)
                v0 = base + c * NB

                @pl.when(jnp.logical_not(valid))
                def _():
                    pltpu.sync_copy(zer_v, out_hbm.at[pl.ds(v0, NB)])

                @pl.when(valid)
                def _():
                    pltpu.make_async_copy(
                        table_hbm.at[idx_bufs[b]], row_bufs[b], sems[b]).wait()
                    compute_chunk(row_bufs[b], w_bufs[b], v0)

                @pl.when(c + 2 < NCH)
                def _prefetch():
                    stage(c + 2, b)
            return carry

        lax.fori_loop(0, NCH // 2, super_body, 0)

    return k(table, idx_flat, w_flat)


def _tr_body(f_ref, t_ref):
    x = f_ref[0].reshape(D // 2, N)
    t_ref[...] = jnp.transpose(x, (1, 0))


def _tc_table(feats):
    return pl.pallas_call(
        _tr_body,
        grid=(B, 2),
        in_specs=[pl.BlockSpec((1, C // 2, A, N), lambda b, j: (b, j, 0, 0))],
        out_specs=pl.BlockSpec((N, D // 2), lambda b, j: (b, j)),
        out_shape=jax.ShapeDtypeStruct((B * N, D), jnp.float32),
    )(feats)


_RB = 512
_GRID = TOT // _RB


def _mm_body(x_ref, wt_ref, y_ref, sum_ref, sq_ref, w2_ref):
    @pl.when(pl.program_id(0) == 0)
    def _init():
        sum_ref[...] = jnp.zeros_like(sum_ref)
        sq_ref[...] = jnp.zeros_like(sq_ref)
        r4 = (lax.broadcasted_iota(jnp.int32, (D, C), 0) // A
              == lax.broadcasted_iota(jnp.int32, (D, C), 1)
              ).astype(jnp.float32)
        mask = (lax.broadcasted_iota(jnp.int32, (D, D), 0) % A
                == lax.broadcasted_iota(jnp.int32, (D, D), 1) % A
                ).astype(jnp.float32)
        for k in range(K):
            rep = jnp.dot(jnp.dot(r4, wt_ref[k],
                                  preferred_element_type=jnp.float32),
                          jnp.transpose(r4, (1, 0)),
                          preferred_element_type=jnp.float32)
            w2_ref[k] = rep * mask

    y = jnp.dot(x_ref[:, 0:D], w2_ref[0], preferred_element_type=jnp.float32)
    for k in range(1, K):
        y += jnp.dot(x_ref[:, k * D:(k + 1) * D], w2_ref[k],
                     preferred_element_type=jnp.float32)
    y_ref[...] = y
    sum_ref[...] += jnp.sum(y, axis=0, keepdims=True)
    sq_ref[...] += jnp.sum(y * y, axis=0, keepdims=True)


def _tc_matmul_stats(x, wt):
    return pl.pallas_call(
        _mm_body,
        grid=(_GRID,),
        in_specs=[
            pl.BlockSpec((_RB, KD), lambda i: (i, 0)),
            pl.BlockSpec((K, C, O), lambda i: (0, 0, 0)),
        ],
        out_specs=[
            pl.BlockSpec((_RB, D), lambda i: (i, 0)),
            pl.BlockSpec((1, D), lambda i: (0, 0)),
            pl.BlockSpec((1, D), lambda i: (0, 0)),
        ],
        out_shape=[
            jax.ShapeDtypeStruct((TOT, D), jnp.float32),
            jax.ShapeDtypeStruct((1, D), jnp.float32),
            jax.ShapeDtypeStruct((1, D), jnp.float32),
        ],
        scratch_shapes=[pltpu.VMEM((K, D, D), jnp.float32)],
    )(x, wt)


def _bn_body(y_ref, sc_ref, sh_ref, o_ref):
    z = jnp.maximum(y_ref[...] * sc_ref[...] + sh_ref[...], 0.0)
    t = jnp.transpose(z, (1, 0))
    o_ref[0] = t.reshape(O, A, MP)[:, :, :M]


def _tc_bn_relu(y, scale_col, shift_col):
    return pl.pallas_call(
        _bn_body,
        grid=(B,),
        in_specs=[
            pl.BlockSpec((MP, D), lambda b: (b, 0)),
            pl.BlockSpec((1, D), lambda b: (0, 0)),
            pl.BlockSpec((1, D), lambda b: (0, 0)),
        ],
        out_specs=pl.BlockSpec((1, O, A, M), lambda b: (b, 0, 0, 0)),
        out_shape=jax.ShapeDtypeStruct((B, O, A, M), jnp.float32),
    )(y, scale_col, shift_col)


def kernel(xyz, feats, inter_idx, inter_w, W, gamma, beta):
    table = _tc_table(feats)
    idxp = (inter_idx.astype(jnp.int32)
            + (jnp.arange(B, dtype=jnp.int32) * N)[:, None, None])
    idx_flat = idxp.reshape(B * M * NN)
    w_flat = inter_w.reshape(B * M, K, NN)
    wt = W.transpose(2, 1, 0)

    wsum = _sc_wsum(table, idx_flat, w_flat)

    y, colsum, colsq = _tc_matmul_stats(wsum, wt)

    cnt = jnp.float32(B * M * A)
    mean = colsum.reshape(O, A).sum(1) / cnt
    var = colsq.reshape(O, A).sum(1) / cnt - mean * mean
    scale_o = gamma * lax.rsqrt(var + 1e-5)
    shift_o = beta - mean * scale_o
    scale_col = jnp.repeat(scale_o, A).reshape(1, D)
    shift_col = jnp.repeat(shift_o, A).reshape(1, D)

    feat = _tc_bn_relu(y, scale_col, shift_col)
    new_xyz = xyz[:, :, ::4]
    return (inter_idx, inter_w, new_xyz, feat)


# speedup vs baseline: 1.5246x; 1.0376x over previous
import functools

import jax
import jax.numpy as jnp
from jax import lax
from jax.experimental import pallas as pl
from jax.experimental.pallas import tpu as pltpu
from jax.experimental.pallas import tpu_sc as plsc

B = 4
N = 10000
M = 2500
MP = 2560
NN = 16
K = 3
A = 4
C = 64
O = 64
D = A * C
TOT = B * MP
KD = K * D

NW = 32
PER_W = TOT // NW
NB = 4
NCH = PER_W // NB
ROWS = NB * NN


def _sc_wsum(table, idx_flat, w_flat):
    mesh = plsc.VectorSubcoreMesh(core_axis_name="c", subcore_axis_name="s")
    info = plsc.get_sparse_core_info()
    nc = info.num_cores

    @functools.partial(
        pl.kernel,
        mesh=mesh,
        out_type=jax.ShapeDtypeStruct((TOT, KD), jnp.float32),
        scratch_types=[
            pltpu.VMEM((ROWS,), jnp.int32),
            pltpu.VMEM((ROWS,), jnp.int32),
            pltpu.VMEM((NB, K, 16), jnp.float32),
            pltpu.VMEM((NB, K, 16), jnp.float32),
            pltpu.VMEM((ROWS, D), jnp.float32),
            pltpu.VMEM((ROWS, D), jnp.float32),
            pltpu.VMEM((NB, KD), jnp.float32),
            pltpu.VMEM((NB, KD), jnp.float32),
            pltpu.VMEM((NB, KD), jnp.float32),
            pltpu.SemaphoreType.DMA,
            pltpu.SemaphoreType.DMA,
            pltpu.SemaphoreType.DMA,
            pltpu.SemaphoreType.DMA,
        ],
    )
    def k(table_hbm, idx_hbm, w_hbm, out_hbm,
          idx_v0, idx_v1, w_v0, w_v1, rows_v0, rows_v1, out_v0, out_v1,
          zer_v, sem0, sem1, semo0, semo1):
        wid = lax.axis_index("s") * nc + lax.axis_index("c")
        base = wid * PER_W
        idx_bufs = (idx_v0, idx_v1)
        w_bufs = (w_v0, w_v1)
        row_bufs = (rows_v0, rows_v1)
        out_bufs = (out_v0, out_v1)
        sems = (sem0, sem1)
        semo = (semo0, semo1)

        for j in range(NB):
            for t in range(KD // 16):
                zer_v[j, pl.ds(t * 16, 16)] = jnp.zeros((16,), jnp.float32)

        def vmap_chunk(c):
            v0 = base + c * NB
            bq = v0 // MP
            mm = v0 - bq * MP
            return mm < M, bq * M + mm

        def stage(c, b):
            valid, r0 = vmap_chunk(c)

            @pl.when(valid)
            def _():
                pltpu.sync_copy(idx_hbm.at[pl.ds(r0 * NN, ROWS)], idx_bufs[b])
                pltpu.sync_copy(w_hbm.at[pl.ds(r0, NB)], w_bufs[b])
                pltpu.make_async_copy(
                    table_hbm.at[idx_bufs[b]], row_bufs[b], sems[b]).start()

        stage(0, 0)
        stage(1, 1)

        def compute_chunk(rows_v, w_v, out_v, v0):
            def j_body(j, carry2):
                wv = [w_v[j, kk, :] for kk in range(K)]
                for dg in range(4):
                    accs = [[jnp.zeros((16,), jnp.float32)
                             for _ in range(4)] for _ in range(K)]
                    for n in range(16):
                        r = [rows_v[j * NN + n, pl.ds(dg * 64 + d * 16, 16)]
                             for d in range(4)]
                        for kk in range(K):
                            wspl = wv[kk].at[
                                jnp.full((16,), n, jnp.int32)
                            ].get(mode="promise_in_bounds")
                            for d in range(4):
                                accs[kk][d] = accs[kk][d] + wspl * r[d]
                    for kk in range(K):
                        for d in range(4):
                            out_v[j, pl.ds(kk * D + dg * 64 + d * 16,
                                           16)] = accs[kk][d]
                return carry2

            lax.fori_loop(0, NB, j_body, 0)

        def super_body(s, carry):
            for b in range(2):
                c = s * 2 + b
                valid, _r0 = vmap_chunk(c)
                v0 = base + c * NB

                @pl.when(jnp.logical_not(valid))
                def _():
                    pltpu.sync_copy(zer_v, out_hbm.at[pl.ds(v0, NB)])

                @pl.when(valid)
                def _():
                    pltpu.make_async_copy(
                        table_hbm.at[idx_bufs[b]], row_bufs[b], sems[b]).wait()

                    @pl.when(c >= 2)
                    def _():
                        pltpu.make_async_copy(
                            out_bufs[b], out_hbm.at[pl.ds(v0, NB)],
                            semo[b]).wait()

                    compute_chunk(row_bufs[b], w_bufs[b], out_bufs[b], v0)
                    pltpu.make_async_copy(
                        out_bufs[b], out_hbm.at[pl.ds(v0, NB)],
                        semo[b]).start()

                @pl.when(c + 2 < NCH)
                def _prefetch():
                    stage(c + 2, b)
            return carry

        lax.fori_loop(0, NCH // 2, super_body, 0)

        for b in range(2):
            pltpu.make_async_copy(
                out_bufs[b], out_hbm.at[pl.ds(base, NB)], semo[b]).wait()

    return k(table, idx_flat, w_flat)


def _tr_body(f_ref, t_ref):
    x = f_ref[0].reshape(D // 2, N)
    t_ref[...] = jnp.transpose(x, (1, 0))


def _tc_table(feats):
    return pl.pallas_call(
        _tr_body,
        grid=(B, 2),
        in_specs=[pl.BlockSpec((1, C // 2, A, N), lambda b, j: (b, j, 0, 0))],
        out_specs=pl.BlockSpec((N, D // 2), lambda b, j: (b, j)),
        out_shape=jax.ShapeDtypeStruct((B * N, D), jnp.float32),
    )(feats)


_RB = 512
_GRID = TOT // _RB


def _mm_body(x_ref, wt_ref, y_ref, sum_ref, sq_ref, w2_ref):
    @pl.when(pl.program_id(0) == 0)
    def _init():
        sum_ref[...] = jnp.zeros_like(sum_ref)
        sq_ref[...] = jnp.zeros_like(sq_ref)
        r4 = (lax.broadcasted_iota(jnp.int32, (D, C), 0) // A
              == lax.broadcasted_iota(jnp.int32, (D, C), 1)
              ).astype(jnp.float32)
        mask = (lax.broadcasted_iota(jnp.int32, (D, D), 0) % A
                == lax.broadcasted_iota(jnp.int32, (D, D), 1) % A
                ).astype(jnp.float32)
        for k in range(K):
            rep = jnp.dot(jnp.dot(r4, wt_ref[k],
                                  preferred_element_type=jnp.float32),
                          jnp.transpose(r4, (1, 0)),
                          preferred_element_type=jnp.float32)
            w2_ref[k] = rep * mask

    y = jnp.dot(x_ref[:, 0:D], w2_ref[0], preferred_element_type=jnp.float32)
    for k in range(1, K):
        y += jnp.dot(x_ref[:, k * D:(k + 1) * D], w2_ref[k],
                     preferred_element_type=jnp.float32)
    y_ref[...] = y
    sum_ref[...] += jnp.sum(y, axis=0, keepdims=True)
    sq_ref[...] += jnp.sum(y * y, axis=0, keepdims=True)


def _tc_matmul_stats(x, wt):
    return pl.pallas_call(
        _mm_body,
        grid=(_GRID,),
        in_specs=[
            pl.BlockSpec((_RB, KD), lambda i: (i, 0)),
            pl.BlockSpec((K, C, O), lambda i: (0, 0, 0)),
        ],
        out_specs=[
            pl.BlockSpec((_RB, D), lambda i: (i, 0)),
            pl.BlockSpec((1, D), lambda i: (0, 0)),
            pl.BlockSpec((1, D), lambda i: (0, 0)),
        ],
        out_shape=[
            jax.ShapeDtypeStruct((TOT, D), jnp.float32),
            jax.ShapeDtypeStruct((1, D), jnp.float32),
            jax.ShapeDtypeStruct((1, D), jnp.float32),
        ],
        scratch_shapes=[pltpu.VMEM((K, D, D), jnp.float32)],
    )(x, wt)


def _bn_body(y_ref, sc_ref, sh_ref, o_ref):
    z = jnp.maximum(y_ref[...] * sc_ref[...] + sh_ref[...], 0.0)
    t = jnp.transpose(z, (1, 0))
    o_ref[0] = t.reshape(O, A, MP)[:, :, :M]


def _tc_bn_relu(y, scale_col, shift_col):
    return pl.pallas_call(
        _bn_body,
        grid=(B,),
        in_specs=[
            pl.BlockSpec((MP, D), lambda b: (b, 0)),
            pl.BlockSpec((1, D), lambda b: (0, 0)),
            pl.BlockSpec((1, D), lambda b: (0, 0)),
        ],
        out_specs=pl.BlockSpec((1, O, A, M), lambda b: (b, 0, 0, 0)),
        out_shape=jax.ShapeDtypeStruct((B, O, A, M), jnp.float32),
    )(y, scale_col, shift_col)


def kernel(xyz, feats, inter_idx, inter_w, W, gamma, beta):
    table = _tc_table(feats)
    idxp = (inter_idx.astype(jnp.int32)
            + (jnp.arange(B, dtype=jnp.int32) * N)[:, None, None])
    idx_flat = idxp.reshape(B * M * NN)
    w_flat = inter_w.reshape(B * M, K, NN)
    wt = W.transpose(2, 1, 0)

    wsum = _sc_wsum(table, idx_flat, w_flat)

    y, colsum, colsq = _tc_matmul_stats(wsum, wt)

    cnt = jnp.float32(B * M * A)
    mean = colsum.reshape(O, A).sum(1) / cnt
    var = colsq.reshape(O, A).sum(1) / cnt - mean * mean
    scale_o = gamma * lax.rsqrt(var + 1e-5)
    shift_o = beta - mean * scale_o
    scale_col = jnp.repeat(scale_o, A).reshape(1, D)
    shift_col = jnp.repeat(shift_o, A).reshape(1, D)

    feat = _tc_bn_relu(y, scale_col, shift_col)
    new_xyz = xyz[:, :, ::4]
    return (inter_idx, inter_w, new_xyz, feat)


# speedup vs baseline: 1.6770x; 1.1000x over previous
import functools

import jax
import jax.numpy as jnp
from jax import lax
from jax.experimental import pallas as pl
from jax.experimental.pallas import tpu as pltpu
from jax.experimental.pallas import tpu_sc as plsc

B = 4
N = 10000
M = 2500
MP = 2560
NN = 16
K = 3
A = 4
C = 64
O = 64
D = A * C
TOT = B * MP
KD = K * D

NW = 32
PER_W = TOT // NW
NB = 4
NCH = PER_W // NB
ROWS = NB * NN


def _sc_wsum(table, idx_flat, w_flat):
    mesh = plsc.VectorSubcoreMesh(core_axis_name="c", subcore_axis_name="s")
    info = plsc.get_sparse_core_info()
    nc = info.num_cores

    @functools.partial(
        pl.kernel,
        mesh=mesh,
        out_type=jax.ShapeDtypeStruct((TOT, KD), jnp.float32),
        scratch_types=[
            pltpu.VMEM((ROWS,), jnp.int32),
            pltpu.VMEM((ROWS,), jnp.int32),
            pltpu.VMEM((PER_W * K * NN,), jnp.float32),
            pltpu.VMEM((ROWS, D), jnp.float32),
            pltpu.VMEM((ROWS, D), jnp.float32),
            pltpu.VMEM((NB, KD), jnp.float32),
            pltpu.VMEM((NB, KD), jnp.float32),
            pltpu.VMEM((NB, KD), jnp.float32),
            pltpu.SemaphoreType.DMA,
            pltpu.SemaphoreType.DMA,
            pltpu.SemaphoreType.DMA,
            pltpu.SemaphoreType.DMA,
        ],
    )
    def k(table_hbm, idx_hbm, w_hbm, out_hbm,
          idx_v0, idx_v1, w_all, rows_v0, rows_v1, out_v0, out_v1,
          zer_v, sem0, sem1, semo0, semo1):
        wid = lax.axis_index("s") * nc + lax.axis_index("c")
        base = wid * PER_W
        idx_bufs = (idx_v0, idx_v1)
        row_bufs = (rows_v0, rows_v1)
        out_bufs = (out_v0, out_v1)
        sems = (sem0, sem1)
        semo = (semo0, semo1)

        bq0 = base // MP
        mm0 = base - bq0 * MP
        rbase = bq0 * M + mm0
        rstart = jnp.minimum(rbase, B * M - PER_W)
        delta = rbase - rstart
        pltpu.sync_copy(
            w_hbm.at[pl.ds(rstart * (K * NN), PER_W * K * NN)], w_all)

        for j in range(NB):
            for t in range(KD // 16):
                zer_v[j, pl.ds(t * 16, 16)] = jnp.zeros((16,), jnp.float32)

        def vmap_chunk(c):
            v0 = base + c * NB
            bq = v0 // MP
            mm = v0 - bq * MP
            return mm < M, bq * M + mm

        def stage(c, b):
            valid, r0 = vmap_chunk(c)

            @pl.when(valid)
            def _():
                pltpu.sync_copy(idx_hbm.at[pl.ds(r0 * NN, ROWS)], idx_bufs[b])
                pltpu.make_async_copy(
                    table_hbm.at[idx_bufs[b]], row_bufs[b], sems[b]).start()

        stage(0, 0)
        stage(1, 1)

        def compute_chunk(rows_v, c, out_v, v0):
            def j_body(j, carry2):
                wv = [w_all[pl.ds((delta + c * NB + j) * (K * NN)
                                  + kk * NN, 16)] for kk in range(K)]
                for dg in range(4):
                    accs = [[jnp.zeros((16,), jnp.float32)
                             for _ in range(4)] for _ in range(K)]
                    for n in range(16):
                        r = [rows_v[j * NN + n, pl.ds(dg * 64 + d * 16, 16)]
                             for d in range(4)]
                        for kk in range(K):
                            wspl = wv[kk].at[
                                jnp.full((16,), n, jnp.int32)
                            ].get(mode="promise_in_bounds")
                            for d in range(4):
                                accs[kk][d] = accs[kk][d] + wspl * r[d]
                    for kk in range(K):
                        for d in range(4):
                            out_v[j, pl.ds(kk * D + dg * 64 + d * 16,
                                           16)] = accs[kk][d]
                return carry2

            lax.fori_loop(0, NB, j_body, 0)

        def super_body(s, carry):
            for b in range(2):
                c = s * 2 + b
                valid, _r0 = vmap_chunk(c)
                v0 = base + c * NB

                @pl.when(jnp.logical_not(valid))
                def _():
                    pltpu.sync_copy(zer_v, out_hbm.at[pl.ds(v0, NB)])

                @pl.when(valid)
                def _():
                    pltpu.make_async_copy(
                        table_hbm.at[idx_bufs[b]], row_bufs[b], sems[b]).wait()

                    @pl.when(c >= 2)
                    def _():
                        pltpu.make_async_copy(
                            out_bufs[b], out_hbm.at[pl.ds(v0, NB)],
                            semo[b]).wait()

                    compute_chunk(row_bufs[b], c, out_bufs[b], v0)
                    pltpu.make_async_copy(
                        out_bufs[b], out_hbm.at[pl.ds(v0, NB)],
                        semo[b]).start()

                @pl.when(c + 2 < NCH)
                def _prefetch():
                    stage(c + 2, b)
            return carry

        lax.fori_loop(0, NCH // 2, super_body, 0)

        for b in range(2):
            pltpu.make_async_copy(
                out_bufs[b], out_hbm.at[pl.ds(base, NB)], semo[b]).wait()

    return k(table, idx_flat, w_flat)


def _tr_body(f_ref, t_ref):
    x = f_ref[0].reshape(D // 2, N)
    t_ref[...] = jnp.transpose(x, (1, 0))


def _tc_table(feats):
    return pl.pallas_call(
        _tr_body,
        grid=(B, 2),
        in_specs=[pl.BlockSpec((1, C // 2, A, N), lambda b, j: (b, j, 0, 0))],
        out_specs=pl.BlockSpec((N, D // 2), lambda b, j: (b, j)),
        out_shape=jax.ShapeDtypeStruct((B * N, D), jnp.float32),
    )(feats)


_RB = 512
_GRID = TOT // _RB


def _mm_body(x_ref, wt_ref, y_ref, sum_ref, sq_ref, w2_ref):
    @pl.when(pl.program_id(0) == 0)
    def _init():
        sum_ref[...] = jnp.zeros_like(sum_ref)
        sq_ref[...] = jnp.zeros_like(sq_ref)
        r4 = (lax.broadcasted_iota(jnp.int32, (D, C), 0) // A
              == lax.broadcasted_iota(jnp.int32, (D, C), 1)
              ).astype(jnp.float32)
        mask = (lax.broadcasted_iota(jnp.int32, (D, D), 0) % A
                == lax.broadcasted_iota(jnp.int32, (D, D), 1) % A
                ).astype(jnp.float32)
        for k in range(K):
            rep = jnp.dot(jnp.dot(r4, wt_ref[k],
                                  preferred_element_type=jnp.float32),
                          jnp.transpose(r4, (1, 0)),
                          preferred_element_type=jnp.float32)
            w2_ref[k] = rep * mask

    y = jnp.dot(x_ref[:, 0:D], w2_ref[0], preferred_element_type=jnp.float32)
    for k in range(1, K):
        y += jnp.dot(x_ref[:, k * D:(k + 1) * D], w2_ref[k],
                     preferred_element_type=jnp.float32)
    y_ref[...] = y
    sum_ref[...] += jnp.sum(y, axis=0, keepdims=True)
    sq_ref[...] += jnp.sum(y * y, axis=0, keepdims=True)


def _tc_matmul_stats(x, wt):
    return pl.pallas_call(
        _mm_body,
        grid=(_GRID,),
        in_specs=[
            pl.BlockSpec((_RB, KD), lambda i: (i, 0)),
            pl.BlockSpec((K, C, O), lambda i: (0, 0, 0)),
        ],
        out_specs=[
            pl.BlockSpec((_RB, D), lambda i: (i, 0)),
            pl.BlockSpec((1, D), lambda i: (0, 0)),
            pl.BlockSpec((1, D), lambda i: (0, 0)),
        ],
        out_shape=[
            jax.ShapeDtypeStruct((TOT, D), jnp.float32),
            jax.ShapeDtypeStruct((1, D), jnp.float32),
            jax.ShapeDtypeStruct((1, D), jnp.float32),
        ],
        scratch_shapes=[pltpu.VMEM((K, D, D), jnp.float32)],
    )(x, wt)


def _bn_body(y_ref, sc_ref, sh_ref, o_ref):
    z = jnp.maximum(y_ref[...] * sc_ref[...] + sh_ref[...], 0.0)
    t = jnp.transpose(z, (1, 0))
    o_ref[0] = t.reshape(O, A, MP)[:, :, :M]


def _tc_bn_relu(y, scale_col, shift_col):
    return pl.pallas_call(
        _bn_body,
        grid=(B,),
        in_specs=[
            pl.BlockSpec((MP, D), lambda b: (b, 0)),
            pl.BlockSpec((1, D), lambda b: (0, 0)),
            pl.BlockSpec((1, D), lambda b: (0, 0)),
        ],
        out_specs=pl.BlockSpec((1, O, A, M), lambda b: (b, 0, 0, 0)),
        out_shape=jax.ShapeDtypeStruct((B, O, A, M), jnp.float32),
    )(y, scale_col, shift_col)


def kernel(xyz, feats, inter_idx, inter_w, W, gamma, beta):
    table = _tc_table(feats)
    idxp = (inter_idx.astype(jnp.int32)
            + (jnp.arange(B, dtype=jnp.int32) * N)[:, None, None])
    idx_flat = idxp.reshape(B * M * NN)
    w_flat = inter_w.reshape(B * M * K * NN)
    wt = W.transpose(2, 1, 0)

    wsum = _sc_wsum(table, idx_flat, w_flat)

    y, colsum, colsq = _tc_matmul_stats(wsum, wt)

    cnt = jnp.float32(B * M * A)
    mean = colsum.reshape(O, A).sum(1) / cnt
    var = colsq.reshape(O, A).sum(1) / cnt - mean * mean
    scale_o = gamma * lax.rsqrt(var + 1e-5)
    shift_o = beta - mean * scale_o
    scale_col = jnp.repeat(scale_o, A).reshape(1, D)
    shift_col = jnp.repeat(shift_o, A).reshape(1, D)

    feat = _tc_bn_relu(y, scale_col, shift_col)
    new_xyz = xyz[:, :, ::4]
    return (inter_idx, inter_w, new_xyz, feat)
